# baseline (speedup 1.0000x reference)
import functools

import jax
import jax.numpy as jnp
import numpy as np
from jax import lax
from jax.experimental import pallas as pl
from jax.experimental.pallas import tpu as pltpu
from jax.experimental.pallas import tpu_sc as plsc

_N_SLOTS = 100000
_D_STATE = 8
_NGRAM = 16
_N_BLADES = 8
_B = 16
_S = 4096
_N_TOKENS = _B * _S
_N_WORKERS = 32
_TOK_PER_W = _N_TOKENS // _N_WORKERS
_GROUPS = _TOK_PER_W // 16
_TOK_PER_DESC = 128
_N_DESC = _TOK_PER_W // _TOK_PER_DESC

_DESC_PER_SUPER = 2
_SUPERS = _N_DESC // _DESC_PER_SUPER
_TOK_PER_SUPER = _DESC_PER_SUPER * _TOK_PER_DESC

_FNV_INIT = np.uint32(2166136261)
_FNV_PRIME = np.uint32(16777619)

_SLOT_TILES = 782
_SLOTS_PAD = _SLOT_TILES * 128


def _bank_tr_body(in_ref, out_ref):
    x = in_ref[...]
    y = x.reshape(64, 128)
    row = lax.broadcasted_iota(jnp.int32, (128, 128), 0)
    col = lax.broadcasted_iota(jnp.int32, (128, 128), 1)
    eye = (row == col).astype(jnp.float32)
    z = lax.dot_general(eye, y, (((0,), (1,)), ((), ())),
                        preferred_element_type=jnp.float32)
    sel = lax.broadcasted_iota(jnp.int32, (64, 128), 0)
    kcol = lax.broadcasted_iota(jnp.int32, (64, 128), 1)
    sel_even = (kcol == 2 * sel).astype(jnp.float32)
    sel_odd = (kcol == 2 * sel + 1).astype(jnp.float32)
    out_ref[...] = jnp.concatenate(
        [lax.dot_general(sel_even, z, (((1,), (0,)), ((), ())),
                         preferred_element_type=jnp.float32),
         lax.dot_general(sel_odd, z, (((1,), (0,)), ((), ())),
                         preferred_element_type=jnp.float32)], axis=1)


def _bank_transpose(bank_p):
    return pl.pallas_call(
        _bank_tr_body,
        grid=(_SLOT_TILES,),
        in_specs=[pl.BlockSpec((_N_BLADES, _D_STATE, 128), lambda b: (0, 0, b))],
        out_specs=pl.BlockSpec((64, 128), lambda b: (b, 0)),
        out_shape=jax.ShapeDtypeStruct((_SLOT_TILES * 64, 128), jnp.float32),
    )(bank_p)


@functools.partial(
    pl.kernel,
    out_type=jax.ShapeDtypeStruct((_B * _N_BLADES, _S // 128, _D_STATE, 128),
                                  jnp.float32),
    mesh=plsc.VectorSubcoreMesh(core_axis_name="c", subcore_axis_name="s"),
    scratch_types=[
        pltpu.VMEM((2, 16, 8, 128), jnp.int32),
        pltpu.VMEM((_N_DESC, _TOK_PER_DESC), jnp.int32),
        pltpu.VMEM((_TOK_PER_SUPER, _N_BLADES * _D_STATE), jnp.float32),
        pltpu.VMEM((_TOK_PER_SUPER, _N_BLADES * _D_STATE), jnp.float32),
        pltpu.VMEM((_N_BLADES, _DESC_PER_SUPER, _D_STATE, 128), jnp.float32),
        pltpu.VMEM((_N_BLADES, _DESC_PER_SUPER, _D_STATE, 128), jnp.float32),
        pltpu.SemaphoreType.DMA,
        pltpu.SemaphoreType.DMA,
        pltpu.SemaphoreType.DMA,
        pltpu.SemaphoreType.DMA,
    ],
    compiler_params=pltpu.CompilerParams(
        needs_layout_passes=False, use_tc_tiling_on_sc=False),
)
def _sc_gather(bw_hbm, bank_hbm, out_hbm, bw_v, idx_v, g_a, g_b, st_a, st_b,
               sem_a, sem_b, sem_oa, sem_ob):
    wid = lax.axis_index("s") * 2 + lax.axis_index("c")
    batch = wid // 2
    shalf = wid % 2
    c0 = shalf * 16
    lanes = lax.iota(jnp.int32, 16)

    pltpu.sync_copy(bw_hbm.at[pl.ds(batch * 2, 2), pl.ds(c0, 16)], bw_v)

    def hash_body(g, carry):
        c = g // 8
        off = (g % 8) * 16
        h = jnp.full((16,), _FNV_INIT, dtype=jnp.uint32)
        for i in range(_NGRAM):
            b = bw_v[i // 8, c, i % 8, pl.ds(off, 16)]
            h = (h ^ plsc.bitcast(b, jnp.uint32)) * _FNV_PRIME
        addr = plsc.bitcast(h % np.uint32(_N_SLOTS), jnp.int32)
        idx_v[g // 8, pl.ds(off, 16)] = addr
        return carry

    lax.fori_loop(0, _GROUPS, hash_body, 0)

    def fire_super(s, buf, sem):
        for j in range(_DESC_PER_SUPER):
            d = s * _DESC_PER_SUPER + j
            pltpu.async_copy(bank_hbm.at[idx_v.at[d]],
                             buf.at[pl.ds(j * _TOK_PER_DESC, _TOK_PER_DESC)],
                             sem)

    def drain_super(s, buf, sem):
        for j in range(_DESC_PER_SUPER):
            d = s * _DESC_PER_SUPER + j
            pltpu.make_async_copy(
                bank_hbm.at[idx_v.at[d]],
                buf.at[pl.ds(j * _TOK_PER_DESC, _TOK_PER_DESC)],
                sem).wait()

    def transpose_super(buf, st):
        def tr_body(v, carry):
            tv = jnp.full((16,), v, dtype=jnp.int32)
            for cc in range(_DESC_PER_SUPER):
                for b8 in range(8):
                    ti = lanes + (cc * 128 + b8 * 16)
                    vec = plsc.load_gather(buf, [ti, tv])
                    st[v // _D_STATE, cc, v % _D_STATE, pl.ds(b8 * 16, 16)] = vec
            return carry

        lax.fori_loop(0, _N_BLADES * _D_STATE, tr_body, 0)

    def out_copy(s, st, sem):
        return pltpu.make_async_copy(
            st,
            out_hbm.at[pl.ds(batch * _N_BLADES, _N_BLADES),
                       pl.ds(c0 + s * _DESC_PER_SUPER, _DESC_PER_SUPER)],
            sem)

    gs = (g_a, g_b)
    sts = (st_a, st_b)
    sems = (sem_a, sem_b)
    osems = (sem_oa, sem_ob)

    fire_super(0, g_a, sem_a)
    fire_super(1, g_b, sem_b)

    for p in range(2):
        drain_super(p, gs[p], sems[p])
        transpose_super(gs[p], sts[p])
        fire_super(p + 2, gs[p], sems[p])
        out_copy(p, sts[p], osems[p]).start()

    def gather_body(i, carry):
        s = 2 + i * 2
        for p in range(2):
            drain_super(s + p, gs[p], sems[p])
            out_copy(s + p - 2, sts[p], osems[p]).wait()
            transpose_super(gs[p], sts[p])
            fire_super(s + p + 2, gs[p], sems[p])
            out_copy(s + p, sts[p], osems[p]).start()
        return carry

    lax.fori_loop(0, (_SUPERS - 4) // 2, gather_body, 0)

    for p in range(2):
        s = _SUPERS - 2 + p
        drain_super(s, gs[p], sems[p])
        out_copy(s - 2, sts[p], osems[p]).wait()
        transpose_super(gs[p], sts[p])
        out_copy(s, sts[p], osems[p]).start()
    for p in range(2):
        out_copy(_SUPERS - 2 + p, sts[p], osems[p]).wait()


def kernel(byte_window, bank):
    bw4 = (byte_window.transpose(0, 2, 1)
           .reshape(_B, 2, 8, _S // 128, 128)
           .transpose(0, 1, 3, 2, 4)
           .reshape(_B * 2, _S // 128, 8, 128))
    bank_p = jnp.transpose(bank, (0, 2, 1))
    bank_lin = _bank_transpose(bank_p).reshape(_SLOTS_PAD, 64)
    out5 = _sc_gather(bw4, bank_lin)
    return (out5.reshape(_B, _N_BLADES, _S // 128, _D_STATE, 128)
            .transpose(0, 2, 4, 1, 3)
            .reshape(_B, _S, _N_BLADES, _D_STATE))


# speedup vs baseline: 2.5010x; 2.5010x over previous
import functools

import jax
import jax.numpy as jnp
import numpy as np
from jax import lax
from jax.experimental import pallas as pl
from jax.experimental.pallas import tpu as pltpu
from jax.experimental.pallas import tpu_sc as plsc

_N_SLOTS = 100000
_D_STATE = 8
_NGRAM = 16
_N_BLADES = 8
_B = 16
_S = 4096
_N_TOKENS = _B * _S
_N_WORKERS = 32
_TOK_PER_W = _N_TOKENS // _N_WORKERS
_GROUPS = _TOK_PER_W // 16
_TOK_PER_DESC = 128
_N_DESC = _TOK_PER_W // _TOK_PER_DESC

_DESC_PER_SUPER = 2
_SUPERS = _N_DESC // _DESC_PER_SUPER
_TOK_PER_SUPER = _DESC_PER_SUPER * _TOK_PER_DESC

_FNV_INIT = np.uint32(2166136261)
_FNV_PRIME = np.uint32(16777619)

_TR_BLK = 512
_SLOT_BLKS = 196
_SLOTS_PAD = _SLOT_BLKS * _TR_BLK


def _bank_tr_body(in_ref, out_ref):
    x = in_ref[...]
    y = x.reshape(64, _TR_BLK)
    z = jnp.transpose(y, (1, 0))
    z2 = z.reshape(_TR_BLK // 2, 2, 64)
    out_ref[...] = jnp.concatenate([z2[:, 0, :], z2[:, 1, :]], axis=1)


def _bank_transpose(bank_p):
    return pl.pallas_call(
        _bank_tr_body,
        grid=(_SLOT_BLKS,),
        in_specs=[pl.BlockSpec((_N_BLADES, _D_STATE, _TR_BLK),
                               lambda b: (0, 0, b))],
        out_specs=pl.BlockSpec((_TR_BLK // 2, 128), lambda b: (b, 0)),
        out_shape=jax.ShapeDtypeStruct((_SLOT_BLKS * _TR_BLK // 2, 128),
                                       jnp.float32),
    )(bank_p)


@functools.partial(
    pl.kernel,
    out_type=jax.ShapeDtypeStruct((_B * _N_BLADES, _S // 128, _D_STATE, 128),
                                  jnp.float32),
    mesh=plsc.VectorSubcoreMesh(core_axis_name="c", subcore_axis_name="s"),
    scratch_types=[
        pltpu.VMEM((2, 16, 8, 128), jnp.int32),
        pltpu.VMEM((_N_DESC, _TOK_PER_DESC), jnp.int32),
        pltpu.VMEM((_TOK_PER_SUPER, _N_BLADES * _D_STATE), jnp.float32),
        pltpu.VMEM((_TOK_PER_SUPER, _N_BLADES * _D_STATE), jnp.float32),
        pltpu.VMEM((_N_BLADES, _DESC_PER_SUPER, _D_STATE, 128), jnp.float32),
        pltpu.VMEM((_N_BLADES, _DESC_PER_SUPER, _D_STATE, 128), jnp.float32),
        pltpu.SemaphoreType.DMA,
        pltpu.SemaphoreType.DMA,
        pltpu.SemaphoreType.DMA,
        pltpu.SemaphoreType.DMA,
    ],
    compiler_params=pltpu.CompilerParams(
        needs_layout_passes=False, use_tc_tiling_on_sc=False),
)
def _sc_gather(bw_hbm, bank_hbm, out_hbm, bw_v, idx_v, g_a, g_b, st_a, st_b,
               sem_a, sem_b, sem_oa, sem_ob):
    wid = lax.axis_index("s") * 2 + lax.axis_index("c")
    batch = wid // 2
    shalf = wid % 2
    c0 = shalf * 16
    lanes = lax.iota(jnp.int32, 16)

    pltpu.sync_copy(bw_hbm.at[pl.ds(batch * 2, 2), pl.ds(c0, 16)], bw_v)

    def hash_body(g, carry):
        c = g // 8
        off = (g % 8) * 16
        h = jnp.full((16,), _FNV_INIT, dtype=jnp.uint32)
        for i in range(_NGRAM):
            b = bw_v[i // 8, c, i % 8, pl.ds(off, 16)]
            h = (h ^ plsc.bitcast(b, jnp.uint32)) * _FNV_PRIME
        addr = plsc.bitcast(h % np.uint32(_N_SLOTS), jnp.int32)
        idx_v[g // 8, pl.ds(off, 16)] = addr
        return carry

    lax.fori_loop(0, _GROUPS, hash_body, 0)

    def fire_super(s, buf, sem):
        for j in range(_DESC_PER_SUPER):
            d = s * _DESC_PER_SUPER + j
            pltpu.async_copy(bank_hbm.at[idx_v.at[d]],
                             buf.at[pl.ds(j * _TOK_PER_DESC, _TOK_PER_DESC)],
                             sem)

    def drain_super(s, buf, sem):
        for j in range(_DESC_PER_SUPER):
            d = s * _DESC_PER_SUPER + j
            pltpu.make_async_copy(
                bank_hbm.at[idx_v.at[d]],
                buf.at[pl.ds(j * _TOK_PER_DESC, _TOK_PER_DESC)],
                sem).wait()

    def transpose_super(buf, st):
        def tr_body(v, carry):
            tv = jnp.full((16,), v, dtype=jnp.int32)
            for cc in range(_DESC_PER_SUPER):
                for b8 in range(8):
                    ti = lanes + (cc * 128 + b8 * 16)
                    vec = plsc.load_gather(buf, [ti, tv])
                    st[v // _D_STATE, cc, v % _D_STATE, pl.ds(b8 * 16, 16)] = vec
            return carry

        lax.fori_loop(0, _N_BLADES * _D_STATE, tr_body, 0)

    def out_copy(s, st, sem):
        return pltpu.make_async_copy(
            st,
            out_hbm.at[pl.ds(batch * _N_BLADES, _N_BLADES),
                       pl.ds(c0 + s * _DESC_PER_SUPER, _DESC_PER_SUPER)],
            sem)

    gs = (g_a, g_b)
    sts = (st_a, st_b)
    sems = (sem_a, sem_b)
    osems = (sem_oa, sem_ob)

    fire_super(0, g_a, sem_a)
    fire_super(1, g_b, sem_b)

    for p in range(2):
        drain_super(p, gs[p], sems[p])
        transpose_super(gs[p], sts[p])
        fire_super(p + 2, gs[p], sems[p])
        out_copy(p, sts[p], osems[p]).start()

    def gather_body(i, carry):
        s = 2 + i * 2
        for p in range(2):
            drain_super(s + p, gs[p], sems[p])
            out_copy(s + p - 2, sts[p], osems[p]).wait()
            transpose_super(gs[p], sts[p])
            fire_super(s + p + 2, gs[p], sems[p])
            out_copy(s + p, sts[p], osems[p]).start()
        return carry

    lax.fori_loop(0, (_SUPERS - 4) // 2, gather_body, 0)

    for p in range(2):
        s = _SUPERS - 2 + p
        drain_super(s, gs[p], sems[p])
        out_copy(s - 2, sts[p], osems[p]).wait()
        transpose_super(gs[p], sts[p])
        out_copy(s, sts[p], osems[p]).start()
    for p in range(2):
        out_copy(_SUPERS - 2 + p, sts[p], osems[p]).wait()


def kernel(byte_window, bank):
    bw4 = (byte_window.transpose(0, 2, 1)
           .reshape(_B, 2, 8, _S // 128, 128)
           .transpose(0, 1, 3, 2, 4)
           .reshape(_B * 2, _S // 128, 8, 128))
    bank_p = jnp.transpose(bank, (0, 2, 1))
    bank_lin = _bank_transpose(bank_p).reshape(_SLOTS_PAD, 64)
    out5 = _sc_gather(bw4, bank_lin)
    return (out5.reshape(_B, _N_BLADES, _S // 128, _D_STATE, 128)
            .transpose(0, 2, 4, 1, 3)
            .reshape(_B, _S, _N_BLADES, _D_STATE))
